# baseline (speedup 1.0000x reference)
import functools

import jax
import jax.numpy as jnp
from jax import lax
from jax.experimental import pallas as pl
from jax.experimental.pallas import tpu as pltpu
from jax.experimental.pallas import tpu_sc as plsc

_B = 1024
_D = 128
_H = 16
_W = 16
_HW = _H * _W
_BB = 32


def _sc_gather(table, labels):
    info = plsc.get_sparse_core_info()
    nw = info.num_cores * info.num_subcores
    b_per_w = _B // nw
    mesh = plsc.VectorSubcoreMesh(core_axis_name="c", subcore_axis_name="s")

    @functools.partial(
        pl.kernel,
        mesh=mesh,
        out_type=jax.ShapeDtypeStruct((_B, _D), jnp.float32),
        scratch_types=[
            pltpu.VMEM((b_per_w,), jnp.int32),
            pltpu.VMEM((b_per_w, _D), jnp.float32),
            pltpu.SemaphoreType.DMA,
        ],
    )
    def gather_kernel(table_hbm, idx_hbm, out_hbm, idx_v, rows_v, sem):
        wid = lax.axis_index("s") * info.num_cores + lax.axis_index("c")
        base = wid * b_per_w
        pltpu.sync_copy(idx_hbm.at[pl.ds(base, b_per_w)], idx_v)
        pltpu.async_copy(table_hbm.at[idx_v], rows_v, sem).wait()
        pltpu.sync_copy(rows_v, out_hbm.at[pl.ds(base, b_per_w)])

    return gather_kernel(table, labels)


def _bcast_body(g_ref, out_ref):
    gt = jnp.transpose(g_ref[...], (1, 0))
    for b in range(_BB):
        out_ref[b] = jnp.broadcast_to(gt[:, b : b + 1], (_D, _HW))


def _tc_broadcast(g):
    out3 = pl.pallas_call(
        _bcast_body,
        grid=(_B // _BB,),
        in_specs=[pl.BlockSpec((_BB, _D), lambda i: (i, 0))],
        out_specs=pl.BlockSpec((_BB, _D, _HW), lambda i: (i, 0, 0)),
        out_shape=jax.ShapeDtypeStruct((_B, _D, _HW), jnp.float32),
    )(g)
    return out3


def kernel(class_labels, embedding_table):
    labels = class_labels.astype(jnp.int32)
    g = _sc_gather(embedding_table, labels)
    out3 = _tc_broadcast(g)
    return out3.reshape(_B, _D, _H, _W)


# speedup vs baseline: 2.8481x; 2.8481x over previous
import functools

import jax
import jax.numpy as jnp
from jax import lax
from jax.experimental import pallas as pl
from jax.experimental.pallas import tpu as pltpu
from jax.experimental.pallas import tpu_sc as plsc

_B = 1024
_D = 128
_H = 16
_W = 16
_HW = _H * _W
_BB = 32


def _sc_gather(table, labels):
    info = plsc.get_sparse_core_info()
    nw = info.num_cores * info.num_subcores
    b_per_w = _B // nw
    mesh = plsc.VectorSubcoreMesh(core_axis_name="c", subcore_axis_name="s")

    @functools.partial(
        pl.kernel,
        mesh=mesh,
        out_type=jax.ShapeDtypeStruct((_B, _D), jnp.float32),
        scratch_types=[
            pltpu.VMEM((b_per_w,), jnp.int32),
            pltpu.VMEM((b_per_w, _D), jnp.float32),
            pltpu.SemaphoreType.DMA,
        ],
    )
    def gather_kernel(table_hbm, idx_hbm, out_hbm, idx_v, rows_v, sem):
        wid = lax.axis_index("s") * info.num_cores + lax.axis_index("c")
        base = wid * b_per_w
        pltpu.sync_copy(idx_hbm.at[pl.ds(base, b_per_w)], idx_v)
        pltpu.async_copy(table_hbm.at[idx_v], rows_v, sem).wait()
        pltpu.sync_copy(rows_v, out_hbm.at[pl.ds(base, b_per_w)])

    return gather_kernel(table, labels)


def _bcast_body(g_ref, out_ref):
    gt = jnp.transpose(g_ref[...], (1, 0))
    for b in range(_BB):
        out_ref[b] = jnp.broadcast_to(gt[:, b : b + 1], (_D, _HW))


def _tc_broadcast(g):
    out3 = pl.pallas_call(
        _bcast_body,
        grid=(_B // _BB,),
        in_specs=[pl.BlockSpec((_BB, _D), lambda i: (i, 0))],
        out_specs=pl.BlockSpec((_BB, _D, _HW), lambda i: (i, 0, 0)),
        out_shape=jax.ShapeDtypeStruct((_B, _D, _HW), jnp.float32),
    )(g)
    return out3


def kernel(class_labels, embedding_table):
    labels = class_labels.astype(jnp.int32)
    g = _sc_gather(embedding_table, labels)
    out3 = jnp.broadcast_to(g[:, :, None], (_B, _D, _HW))
    return out3.reshape(_B, _D, _H, _W)
